# Initial kernel scaffold; baseline (speedup 1.0000x reference)
import jax
import jax.numpy as jnp
from jax.experimental import pallas as pl


def kernel(position, sin_values, cos_values):
    raise NotImplementedError("write your pallas kernel here")


# speedup vs baseline: 3.1993x; 3.1993x over previous
import functools

import jax
import jax.numpy as jnp
from jax import lax
from jax.experimental import pallas as pl
from jax.experimental.pallas import tpu as pltpu
from jax.experimental.pallas import tpu_sc as plsc

_B = 4 * 8192
_D = 64
_NC, _NS = 2, 16
_NW = _NC * _NS
_BPW = _B // _NW

_mesh = plsc.VectorSubcoreMesh(core_axis_name="c", subcore_axis_name="s")


@functools.partial(
    pl.kernel,
    mesh=_mesh,
    out_type=(
        jax.ShapeDtypeStruct((_B, _D), jnp.float32),
        jax.ShapeDtypeStruct((_B, _D), jnp.float32),
    ),
    scratch_types=[
        pltpu.VMEM((_BPW,), jnp.int32),
        pltpu.VMEM((_BPW, _D), jnp.float32),
        pltpu.SemaphoreType.DMA,
    ],
    compiler_params=pltpu.CompilerParams(use_tc_tiling_on_sc=False),
)
def _gather_rows(pos_hbm, sin_hbm, cos_hbm, out_sin, out_cos, idx_v, buf_v, sem):
    wid = lax.axis_index("s") * _NC + lax.axis_index("c")
    base = wid * _BPW
    pltpu.sync_copy(pos_hbm.at[pl.ds(base, _BPW)], idx_v)
    pltpu.async_copy(sin_hbm.at[idx_v], buf_v, sem).wait()
    pltpu.sync_copy(buf_v, out_sin.at[pl.ds(base, _BPW)])
    pltpu.async_copy(cos_hbm.at[idx_v], buf_v, sem).wait()
    pltpu.sync_copy(buf_v, out_cos.at[pl.ds(base, _BPW)])


def kernel(position, sin_values, cos_values):
    batch, seq = position.shape
    flat_pos = position.reshape(_B)
    sin_flat, cos_flat = _gather_rows(flat_pos, sin_values, cos_values)
    return (
        sin_flat.reshape(batch, seq, _D),
        cos_flat.reshape(batch, seq, _D),
    )


# speedup vs baseline: 3.2094x; 1.0032x over previous
import functools

import jax
import jax.numpy as jnp
from jax import lax
from jax.experimental import pallas as pl
from jax.experimental.pallas import tpu as pltpu
from jax.experimental.pallas import tpu_sc as plsc

_B = 4 * 8192
_D = 64
_NC, _NS = 2, 16
_NW = _NC * _NS
_BPW = _B // _NW

_mesh = plsc.VectorSubcoreMesh(core_axis_name="c", subcore_axis_name="s")


_CH = 256
_NCH = _BPW // _CH
_NTASK = 2 * _NCH
_NBUF = 4
_DEPTH = 2


@functools.partial(
    pl.kernel,
    mesh=_mesh,
    out_type=(
        jax.ShapeDtypeStruct((_B, _D), jnp.float32),
        jax.ShapeDtypeStruct((_B, _D), jnp.float32),
    ),
    scratch_types=[
        pltpu.VMEM((_BPW,), jnp.int32),
        [pltpu.VMEM((_CH, _D), jnp.float32) for _ in range(_NBUF)],
        pltpu.SemaphoreType.DMA((_NBUF,)),
        pltpu.SemaphoreType.DMA((_NBUF,)),
    ],
    compiler_params=pltpu.CompilerParams(use_tc_tiling_on_sc=False),
)
def _gather_rows(pos_hbm, sin_hbm, cos_hbm, out_sin, out_cos,
                 idx_v, bufs, g_sem, w_sem):
    wid = lax.axis_index("s") * _NC + lax.axis_index("c")
    base = wid * _BPW
    pltpu.sync_copy(pos_hbm.at[pl.ds(base, _BPW)], idx_v)

    tables = (sin_hbm, cos_hbm)
    outs = (out_sin, out_cos)

    def start_gather(t):
        chunk = t // 2
        idx_sl = idx_v.at[pl.ds(chunk * _CH, _CH)]
        return pltpu.async_copy(tables[t % 2].at[idx_sl],
                                bufs[t % _NBUF], g_sem.at[t % _NBUF])

    def start_wb(t):
        chunk = t // 2
        dst = outs[t % 2].at[pl.ds(base + chunk * _CH, _CH)]
        return pltpu.async_copy(bufs[t % _NBUF], dst, w_sem.at[t % _NBUF])

    gathers = {t: start_gather(t) for t in range(_DEPTH)}
    wbs = {}
    for w in range(_NTASK):
        nx = w + _DEPTH
        if nx < _NTASK:
            if nx >= _NBUF:
                wbs[nx - _NBUF].wait()
            gathers[nx] = start_gather(nx)
        gathers[w].wait()
        wbs[w] = start_wb(w)
    for t in range(_NTASK - _NBUF, _NTASK):
        wbs[t].wait()


def kernel(position, sin_values, cos_values):
    batch, seq = position.shape
    flat_pos = position.reshape(_B)
    sin_flat, cos_flat = _gather_rows(flat_pos, sin_values, cos_values)
    return (
        sin_flat.reshape(batch, seq, _D),
        cos_flat.reshape(batch, seq, _D),
    )
